# baseline (speedup 1.0000x reference)
import functools
import jax
import jax.numpy as jnp
from jax import lax
from jax.experimental import pallas as pl
from jax.experimental.pallas import tpu as pltpu
from jax.experimental.pallas import tpu_sc as plsc

DIM = 64
ENCDIM = 64
BATCH = 16384

_NC = 2
_NS = 16
_NW = _NC * _NS
_BPW = BATCH // _NW

_mesh = plsc.VectorSubcoreMesh(core_axis_name="c", subcore_axis_name="s")


@functools.partial(
    pl.kernel,
    mesh=_mesh,
    out_type=jax.ShapeDtypeStruct((BATCH, DIM + ENCDIM), jnp.float32),
    scratch_types=[
        pltpu.VMEM((_BPW,), jnp.int32),
        pltpu.VMEM((_BPW,), jnp.int32),
        pltpu.VMEM((_BPW, DIM), jnp.float32),
        pltpu.VMEM((_BPW, ENCDIM), jnp.float32),
        pltpu.SemaphoreType.DMA,
    ],
    compiler_params=pltpu.CompilerParams(use_tc_tiling_on_sc=False),
)
def _dil_kernel(ids_hbm, prev_hbm, emb_hbm, map_hbm, out_hbm,
                ids_v, mid_v, emb_v, prev_v, sem):
    wid = lax.axis_index("s") * _NC + lax.axis_index("c")
    base = wid * _BPW
    pltpu.sync_copy(ids_hbm.at[pl.ds(base, _BPW)], ids_v)
    pltpu.async_copy(map_hbm.at[ids_v], mid_v, sem).wait()
    emb_cp = pltpu.async_copy(emb_hbm.at[mid_v], emb_v, sem)
    pltpu.sync_copy(prev_hbm.at[pl.ds(base, _BPW)], prev_v)
    emb_cp.wait()
    pltpu.sync_copy(emb_v, out_hbm.at[pl.ds(base, _BPW), pl.ds(0, DIM)])
    pltpu.sync_copy(prev_v, out_hbm.at[pl.ds(base, _BPW), pl.ds(DIM, ENCDIM)])


def kernel(ids, prev_inp_summ, emb_table, mapper):
    return _dil_kernel(ids.astype(jnp.int32), prev_inp_summ, emb_table,
                       mapper.astype(jnp.int32))
